# baseline (speedup 1.0000x reference)
import jax
import jax.numpy as jnp
from jax.experimental import pallas as pl


def _fused_cell(x_ref, w_ref, b_ref, wlin_ref, blin_ref, o_ref, *, out_ch):
    xb = x_ref[...]
    w = w_ref[0] + w_ref[1]
    y = jnp.dot(xb, w, preferred_element_type=jnp.float32) + b_ref[...]
    z = jax.nn.sigmoid(y[:, :out_ch])
    ht = jnp.tanh(y[:, out_ch:])
    h = jnp.maximum((1.0 - z) * ht, 0.0)
    o_ref[...] = (jnp.sum(h * wlin_ref[...], axis=1, keepdims=True)
                  + blin_ref[...])


def kernel(x, edge_index, edge_weight, W_z, b_z, W_r, b_r, W_h, b_h,
           W_lin, b_lin):
    del edge_index, edge_weight, W_r, b_r
    n, in_ch = x.shape
    out_ch = W_z.shape[-1]

    w = jnp.concatenate([W_z[:, 0, :in_ch, :], W_h[:, 0, :in_ch, :]], axis=-1)
    b = jnp.concatenate([b_z, b_h]).reshape(1, 2 * out_ch)
    wlin = W_lin.reshape(1, out_ch)
    blin = b_lin.reshape(1, 1)

    block = 2000
    grid = (n + block - 1) // block

    import functools
    full = lambda i: (0, 0)
    full3 = lambda i: (0, 0, 0)
    return pl.pallas_call(
        functools.partial(_fused_cell, out_ch=out_ch),
        grid=(grid,),
        in_specs=[
            pl.BlockSpec((block, in_ch), lambda i: (i, 0)),
            pl.BlockSpec((2, in_ch, 2 * out_ch), full3),
            pl.BlockSpec((1, 2 * out_ch), full),
            pl.BlockSpec((1, out_ch), full),
            pl.BlockSpec((1, 1), full),
        ],
        out_specs=pl.BlockSpec((block, 1), lambda i: (i, 0)),
        out_shape=jax.ShapeDtypeStruct((n, 1), x.dtype),
    )(x, w, b, wlin, blin)
